# baseline (speedup 1.0000x reference)
import jax
import jax.numpy as jnp
from jax import lax
from jax.experimental import pallas as pl
from jax.experimental.pallas import tpu as pltpu
from jax.experimental.pallas import tpu_sc as plsc

B, H, W, C = 256, 16, 16, 256
NUM_CLASSES = 10
FLAT = H * W * C
SLABS = 2
CH = SLABS * C
NCH = FLAT // CH
NC, NS = 2, 16
NW = NC * NS
RW = B // NW
GROUP = SLABS * W * C // CH
GPR = H // SLABS
NG = RW * GPR
NBUF = 4
PF = NBUF - 1


def _body(t_hbm, lab_hbm, tab_hbm, out_hbm, lab_v, t_buf, e_buf, tab_s,
          sem_t, sem_e, sem_o, sem_s):
    sid = lax.axis_index("s")
    wid = sid * NC + lax.axis_index("c")
    base_row = wid * RW


    def tensor_copy(g, p):
        return pltpu.make_async_copy(
            t_hbm.at[base_row + g // GPR, pl.ds((g % GPR) * SLABS, SLABS)],
            t_buf.at[p], sem_t.at[p])

    def gather_copy(g, p):
        lab = lab_v[pl.ds(base_row + g // GPR, 16)][0]
        return pltpu.make_async_copy(
            tab_s.at[pl.ds(lab * FLAT + (g % GPR) * (GROUP * CH), GROUP * CH)],
            e_buf.at[p], sem_e.at[p])

    def store_copy(g, p):
        return pltpu.make_async_copy(
            t_buf.at[p],
            out_hbm.at[base_row + g // GPR, pl.ds((g % GPR) * SLABS, SLABS)],
            sem_o.at[p])

    for k in range(PF):
        tensor_copy(k, k).start()

    PIECE = FLAT // 8
    stage = []
    for t in range(5):
        piece = sid * 5 + t
        cls = piece // 8
        e = piece % 8
        stage.append(pltpu.make_async_copy(
            tab_hbm.at[cls, pl.ds(e * PIECE, PIECE)],
            tab_s.at[pl.ds(cls * FLAT + e * PIECE, PIECE)], sem_s))
    for c in stage:
        c.start()
    pltpu.sync_copy(lab_hbm, lab_v.at[pl.ds(0, B)])
    for c in stage:
        c.wait()
    plsc.subcore_barrier()

    for k in range(PF):
        gather_copy(k, k).start()

    def group_body(g, carry):
        p = g % NBUF

        tensor_copy(g, p).wait()
        gather_copy(g, p).wait()

        NM = C // 16

        for i in range(SLABS):
            @plsc.parallel_loop(0, W, 1, unroll=4)
            def _j(j, i=i):
                eoff = (i * W + j) * C
                vals = [e_buf[p, pl.ds(eoff + m * 16, 16)]
                        for m in range(NM)]
                for m in range(NM):
                    plsc.addupdate(t_buf.at[p, i, j, pl.ds(m * 16, 16)],
                                   vals[m])

        store_copy(g, p).start()

        @pl.when(jnp.logical_and(g >= 1, g + PF < NG))
        def _():
            store_copy(g - 1, (g - 1) % NBUF).wait()

        @pl.when(g + PF < NG)
        def _():
            tensor_copy(g + PF, (g + PF) % NBUF).start()
            gather_copy(g + PF, (g + PF) % NBUF).start()

        return carry

    lax.fori_loop(0, NG, group_body, None)
    for k in range(NG - NBUF, NG):
        store_copy(k, k % NBUF).wait()


@jax.jit
def _run(tensor, labels, tab2):
    kfn = pl.kernel(
        _body,
        out_type=jax.ShapeDtypeStruct((B, H, W, C), jnp.float32),
        mesh=plsc.VectorSubcoreMesh(core_axis_name="c", subcore_axis_name="s",
                                    num_cores=NC, num_subcores=NS),
        scratch_types=[
            pltpu.VMEM((B + 16,), jnp.int32),
            pltpu.VMEM((NBUF, SLABS, W, C), jnp.float32),
            pltpu.VMEM((NBUF, GROUP * CH), jnp.float32),
            pltpu.VMEM_SHARED((NUM_CLASSES * FLAT,), jnp.float32),
            pltpu.SemaphoreType.DMA((NBUF,)),
            pltpu.SemaphoreType.DMA((NBUF,)),
            pltpu.SemaphoreType.DMA((NBUF,)),
            pltpu.SemaphoreType.DMA,
        ],
        compiler_params=pltpu.CompilerParams(needs_layout_passes=False),
    )
    return kfn(tensor, labels, tab2)


def kernel(tensor, labels, embed_table):
    return _run(tensor, labels.astype(jnp.int32), embed_table)


# speedup vs baseline: 1.0134x; 1.0134x over previous
import jax
import jax.numpy as jnp
from jax import lax
from jax.experimental import pallas as pl
from jax.experimental.pallas import tpu as pltpu
from jax.experimental.pallas import tpu_sc as plsc

B, H, W, C = 256, 16, 16, 256
NUM_CLASSES = 10
FLAT = H * W * C
SLABS = 2
CH = SLABS * C
NCH = FLAT // CH
NC, NS = 2, 16
NW = NC * NS
RW = B // NW
GROUP = SLABS * W * C // CH
GPR = H // SLABS
NG = RW * GPR
NBUF = 4
PF = NBUF - 1


def _body(t_hbm, lab_hbm, tab_hbm, out_hbm, lab_v, t_buf, e_buf, tab_s,
          sem_t, sem_e, sem_o, sem_s):
    sid = lax.axis_index("s")
    wid = sid * NC + lax.axis_index("c")
    base_row = wid * RW


    def tensor_copy(g, p):
        return pltpu.make_async_copy(
            t_hbm.at[base_row + g // GPR, pl.ds((g % GPR) * SLABS, SLABS)],
            t_buf.at[p], sem_t.at[p])

    def gather_copy(g, p):
        lab = lab_v[pl.ds(base_row + g // GPR, 16)][0]
        return pltpu.make_async_copy(
            tab_s.at[pl.ds(lab * FLAT + (g % GPR) * (GROUP * CH), GROUP * CH)],
            e_buf.at[p], sem_e.at[p])

    def store_copy(g, p):
        return pltpu.make_async_copy(
            t_buf.at[p],
            out_hbm.at[base_row + g // GPR, pl.ds((g % GPR) * SLABS, SLABS)],
            sem_o.at[p])

    for k in range(PF):
        tensor_copy(k, k).start()

    PIECE = FLAT // 8
    stage = []
    for t in range(5):
        piece = sid * 5 + t
        cls = piece // 8
        e = piece % 8
        stage.append(pltpu.make_async_copy(
            tab_hbm.at[cls, pl.ds(e * PIECE, PIECE)],
            tab_s.at[pl.ds(cls * FLAT + e * PIECE, PIECE)], sem_s))
    for c in stage:
        c.start()
    pltpu.sync_copy(lab_hbm, lab_v.at[pl.ds(0, B)])
    for c in stage:
        c.wait()
    plsc.subcore_barrier()

    for k in range(PF):
        gather_copy(k, k).start()

    def group_body(g, carry):
        p = g % NBUF

        tensor_copy(g, p).wait()
        gather_copy(g, p).wait()

        NM = C // 16

        for i in range(SLABS):
            @plsc.parallel_loop(0, W, 1, unroll=2)
            def _j(j, i=i):
                eoff = (i * W + j) * C
                vals = [e_buf[p, pl.ds(eoff + m * 16, 16)]
                        for m in range(NM)]
                for m in range(NM):
                    plsc.addupdate(t_buf.at[p, i, j, pl.ds(m * 16, 16)],
                                   vals[m])

        store_copy(g, p).start()

        @pl.when(jnp.logical_and(g >= 1, g + PF < NG))
        def _():
            store_copy(g - 1, (g - 1) % NBUF).wait()

        @pl.when(g + PF < NG)
        def _():
            tensor_copy(g + PF, (g + PF) % NBUF).start()
            gather_copy(g + PF, (g + PF) % NBUF).start()

        return carry

    lax.fori_loop(0, NG, group_body, None)
    for k in range(NG - NBUF, NG):
        store_copy(k, k % NBUF).wait()


@jax.jit
def _run(tensor, labels, tab2):
    kfn = pl.kernel(
        _body,
        out_type=jax.ShapeDtypeStruct((B, H, W, C), jnp.float32),
        mesh=plsc.VectorSubcoreMesh(core_axis_name="c", subcore_axis_name="s",
                                    num_cores=NC, num_subcores=NS),
        scratch_types=[
            pltpu.VMEM((B + 16,), jnp.int32),
            pltpu.VMEM((NBUF, SLABS, W, C), jnp.float32),
            pltpu.VMEM((NBUF, GROUP * CH), jnp.float32),
            pltpu.VMEM_SHARED((NUM_CLASSES * FLAT,), jnp.float32),
            pltpu.SemaphoreType.DMA((NBUF,)),
            pltpu.SemaphoreType.DMA((NBUF,)),
            pltpu.SemaphoreType.DMA((NBUF,)),
            pltpu.SemaphoreType.DMA,
        ],
        compiler_params=pltpu.CompilerParams(needs_layout_passes=False),
    )
    return kfn(tensor, labels, tab2)


def kernel(tensor, labels, embed_table):
    return _run(tensor, labels.astype(jnp.int32), embed_table)
